# baseline (speedup 1.0000x reference)
import jax
import jax.numpy as jnp
from jax.experimental import pallas as pl
from jax.experimental.pallas import tpu as pltpu

N = 10000
D_IN = 128
D_OUT = 128
BM = 400
NBUF = 3
NUM_M = N // BM


def _gcn_kernel(x_ref, w_ref, b_ref, a_ref, adj_ref, out_ref,
                fts_ref, buf_ref, sem_ref):
    m = pl.program_id(0)

    def _copy(block, slot):
        return pltpu.make_async_copy(
            adj_ref.at[pl.ds(block * BM, BM), :],
            buf_ref.at[slot],
            sem_ref.at[slot],
        )

    @pl.when(m == 0)
    def _prologue():
        for i in range(NBUF):
            _copy(i, i).start()
        fts_ref[...] = jax.lax.dot_general(
            x_ref[...], w_ref[...],
            dimension_numbers=(((1,), (1,)), ((), ())),
            preferred_element_type=jnp.float32,
        ).astype(jnp.bfloat16)

    slot = jax.lax.rem(m, NBUF)
    _copy(m, slot).wait()

    r = jnp.dot(
        buf_ref[slot].astype(jnp.bfloat16), fts_ref[...],
        preferred_element_type=jnp.float32,
    ) + b_ref[...]
    out_ref[...] = jnp.where(r >= 0, r, a_ref[0, 0] * r)

    @pl.when(m + NBUF < NUM_M)
    def _prefetch():
        _copy(m + NBUF, slot).start()


@jax.jit
def kernel(x, adj_mat, W, bias, prelu_a):
    x2 = jnp.squeeze(x, 0)
    b2 = bias.reshape(1, D_OUT)
    a2 = prelu_a.reshape(1, 1)

    out = pl.pallas_call(
        _gcn_kernel,
        grid=(NUM_M,),
        in_specs=[
            pl.BlockSpec((N, D_IN), lambda m: (0, 0)),
            pl.BlockSpec((D_OUT, D_IN), lambda m: (0, 0)),
            pl.BlockSpec((1, D_OUT), lambda m: (0, 0)),
            pl.BlockSpec((1, 1), lambda m: (0, 0)),
            pl.BlockSpec(memory_space=pl.ANY),
        ],
        out_specs=pl.BlockSpec((BM, D_OUT), lambda m: (m, 0)),
        out_shape=jax.ShapeDtypeStruct((N, D_OUT), jnp.float32),
        scratch_shapes=[
            pltpu.VMEM((N, D_OUT), jnp.bfloat16),
            pltpu.VMEM((NBUF, BM, N), jnp.float32),
            pltpu.SemaphoreType.DMA((NBUF,)),
        ],
        compiler_params=pltpu.CompilerParams(
            dimension_semantics=("arbitrary",),
        ),
    )(x2, W, b2, a2, adj_mat)

    return out[None, :, :]


# speedup vs baseline: 1.0449x; 1.0449x over previous
import jax
import jax.numpy as jnp
from jax.experimental import pallas as pl
from jax.experimental.pallas import tpu as pltpu

N = 10000
D_IN = 128
D_OUT = 128
BM = 400


def _gcn_kernel(x_ref, w_ref, b_ref, a_ref, adj_ref, out_ref, fts_ref):
    m = pl.program_id(0)

    @pl.when(m == 0)
    def _compute_fts():
        fts_ref[...] = jax.lax.dot_general(
            x_ref[...], w_ref[...],
            dimension_numbers=(((1,), (1,)), ((), ())),
            preferred_element_type=jnp.float32,
        ).astype(jnp.bfloat16)

    r = jnp.dot(
        adj_ref[...].astype(jnp.bfloat16), fts_ref[...],
        preferred_element_type=jnp.float32,
    ) + b_ref[...]
    out_ref[...] = jnp.where(r >= 0, r, a_ref[0, 0] * r)


@jax.jit
def kernel(x, adj_mat, W, bias, prelu_a):
    x2 = jnp.squeeze(x, 0)
    b2 = bias.reshape(1, D_OUT)
    a2 = prelu_a.reshape(1, 1)

    out = pl.pallas_call(
        _gcn_kernel,
        grid=(N // BM,),
        in_specs=[
            pl.BlockSpec((N, D_IN), lambda m: (0, 0)),
            pl.BlockSpec((D_OUT, D_IN), lambda m: (0, 0)),
            pl.BlockSpec((1, D_OUT), lambda m: (0, 0)),
            pl.BlockSpec((1, 1), lambda m: (0, 0)),
            pl.BlockSpec((BM, N), lambda m: (m, 0)),
        ],
        out_specs=pl.BlockSpec((BM, D_OUT), lambda m: (m, 0)),
        out_shape=jax.ShapeDtypeStruct((N, D_OUT), jnp.float32),
        scratch_shapes=[pltpu.VMEM((N, D_OUT), jnp.bfloat16)],
        compiler_params=pltpu.CompilerParams(
            dimension_semantics=("arbitrary",),
        ),
    )(x2, W, b2, a2, adj_mat)

    return out[None, :, :]
